# Initial kernel scaffold; baseline (speedup 1.0000x reference)
import jax
import jax.numpy as jnp
from jax.experimental import pallas as pl


def kernel(x, pos, edge_index, edge_attr, node_attr, batch, additional_message_features, W_embed, W_msg0, W_msg1, W_pp1, W_pp2):
    raise NotImplementedError("write your pallas kernel here")


# speedup vs baseline: 1.5704x; 1.5704x over previous
import functools

import jax
import jax.numpy as jnp
from jax import lax
from jax.experimental import pallas as pl
from jax.experimental.pallas import tpu as pltpu
from jax.experimental.pallas import tpu_sc as plsc

N = 10000
E = 160000
DW = 128
NC = 2
NSUB = 16
NW = NC * NSUB
EPW = 5120
E_PAD = NW * EPW
CHUNK = 128
NCHUNK = EPW // CHUNK
RPT = 624
RCP = 640
BN = 1000
BE = 2048

_HI = jax.lax.Precision.HIGHEST


@functools.cache
def _mesh():
    return plsc.VectorSubcoreMesh(
        core_axis_name="c", subcore_axis_name="s",
        num_cores=NC, num_subcores=NSUB)


def _tp9(xv, av, w, precision=None):
    acc = None
    for j in range(9):
        t = jnp.dot(xv * av[:, j:j + 1], w[j],
                    preferred_element_type=jnp.float32, precision=precision)
        acc = t if acc is None else acc + t
    return acc


def _gate(agg):
    s = agg[:, :16]
    g = agg[:, 16:24]
    v = agg[:, 24:48]
    sg = jax.nn.sigmoid(g)
    rows = lax.broadcasted_iota(jnp.int32, (8, 24), 0)
    cols = lax.broadcasted_iota(jnp.int32, (8, 24), 1)
    expand = jnp.where(cols // 3 == rows, 1.0, 0.0).astype(jnp.float32)
    vscale = jnp.dot(sg, expand, preferred_element_type=jnp.float32,
                     precision=_HI)
    z = jnp.zeros((agg.shape[0], 8), jnp.float32)
    return jnp.concatenate([s * jax.nn.sigmoid(s), v * vscale, z], axis=1)


def _ones_col0(na):
    cols = lax.broadcasted_iota(jnp.int32, na.shape, 1)
    return jnp.where(cols == 0, 1.0, na)


def _padw(x, width=DW):
    b, w = x.shape
    if w == width:
        return x
    return jnp.concatenate([x, jnp.zeros((b, width - w), jnp.float32)], axis=1)


def _embed_body(x_ref, na_ref, w_ref, h_ref):
    na = _ones_col0(na_ref[...])
    x = _padw(x_ref[...])
    h_ref[...] = _padw(_tp9(x, na, w_ref[...]))


def _payload1_body(g_ref, ea_ref, amf_ref, ws_ref, out_ref):
    ea = ea_ref[...]
    amf = amf_ref[...]
    msg = _tp9(g_ref[...], ea, ws_ref[...])
    s0 = amf[:, 0:1] * ea
    s1 = amf[:, 1:2] * ea
    z = jnp.zeros((msg.shape[0], DW - 75), jnp.float32)
    out_ref[...] = jnp.concatenate([msg, ea, s0, s1, z], axis=1)


def _payload2_body(g_ref, ea_ref, ws_ref, out_ref):
    out_ref[...] = _padw(_tp9(g_ref[...], ea_ref[...], ws_ref[...]))


def _update1_body(p0_ref, p1_ref, h_ref, wd_ref, wa_ref, h_out_ref, as_ref):
    r = p0_ref[...] + p1_ref[...]
    h = h_ref[...]
    a = r[:, 48:57]
    s = r[:, 57:75]
    agg = r[:, :48] + _tp9(h, a, wd_ref[...], _HI) + jnp.dot(
        s, wa_ref[...], preferred_element_type=jnp.float32, precision=_HI)
    h_out_ref[...] = h + _padw(_gate(agg))
    as_ref[...] = r[:, 48:75]


def _final_body(p0_ref, p1_ref, h_ref, as_ref, na_ref, wd_ref, wa_ref,
                wp1_ref, wp2_ref, out_ref):
    r = p0_ref[...] + p1_ref[...]
    h = h_ref[...]
    asv = as_ref[...]
    agg = r[:, :48] + _tp9(h, asv[:, :9], wd_ref[...], _HI) + jnp.dot(
        asv[:, 9:27], wa_ref[...], preferred_element_type=jnp.float32,
        precision=_HI)
    h2 = h + _padw(_gate(agg))
    na = _ones_col0(na_ref[...])
    t = _padw(_gate(_tp9(h2, na, wp1_ref[...])))
    out_ref[...] = _tp9(t, na, wp2_ref[...])


@functools.cache
def _sc_gather():
    @functools.partial(
        pl.kernel, mesh=_mesh(),
        out_type=jax.ShapeDtypeStruct((E_PAD, DW), jnp.float32),
        scratch_types=[
            pltpu.VMEM((CHUNK,), jnp.int32),
            pltpu.VMEM((CHUNK, DW), jnp.float32),
            pltpu.SemaphoreType.DMA,
        ])
    def gather(h_hbm, idx_hbm, out_hbm, idx_v, rows_v, sem):
        wid = lax.axis_index("s") * NC + lax.axis_index("c")
        base = wid * EPW

        def body(i, carry):
            off = pl.multiple_of(base + i * CHUNK, CHUNK)
            pltpu.sync_copy(idx_hbm.at[pl.ds(off, CHUNK)], idx_v)
            pltpu.async_copy(h_hbm.at[idx_v], rows_v, sem).wait()
            pltpu.sync_copy(rows_v, out_hbm.at[pl.ds(off, CHUNK)])
            return carry

        lax.fori_loop(0, NCHUNK, body, 0)

    return gather


@functools.cache
def _sc_scatter():
    @functools.partial(
        pl.kernel, mesh=_mesh(),
        out_type=jax.ShapeDtypeStruct((2 * N, DW), jnp.float32),
        scratch_types=[
            pltpu.VMEM((CHUNK,), jnp.int32),
            pltpu.VMEM((CHUNK, DW), jnp.float32),
            pltpu.VMEM_SHARED((N, DW), jnp.float32),
            pltpu.SemaphoreType.DMA,
        ])
    def scatter(pay_hbm, idx_hbm, zeros_hbm, out_hbm, idx_v, rows_v, acc, sem):
        cid = lax.axis_index("c")
        sid = lax.axis_index("s")
        wid = sid * NC + cid
        base = wid * EPW
        r0 = sid * RPT
        pltpu.sync_copy(zeros_hbm, acc.at[pl.ds(r0, RCP)])
        plsc.subcore_barrier()

        def body(i, carry):
            off = pl.multiple_of(base + i * CHUNK, CHUNK)
            pltpu.sync_copy(idx_hbm.at[pl.ds(off, CHUNK)], idx_v)
            pltpu.sync_copy(pay_hbm.at[pl.ds(off, CHUNK)], rows_v)
            pltpu.sync_copy(rows_v, acc.at[idx_v], add=True)
            return carry

        lax.fori_loop(0, NCHUNK, body, 0)
        plsc.subcore_barrier()
        pltpu.sync_copy(acc.at[pl.ds(r0, RCP)],
                        out_hbm.at[pl.ds(cid * N + r0, RCP)])

    return scatter


def _node_spec(w, i_map=None):
    return pl.BlockSpec((BN, w), i_map or (lambda i: (i, 0)))


def _full_spec(shape):
    nd = len(shape)
    return pl.BlockSpec(shape, lambda i: (0,) * nd)


def _embed(x, na, we):
    return pl.pallas_call(
        _embed_body,
        grid=(N // BN,),
        in_specs=[_node_spec(16), _node_spec(9), _full_spec((9, DW, 48))],
        out_specs=_node_spec(DW),
        out_shape=jax.ShapeDtypeStruct((N, DW), jnp.float32),
    )(x, na, we)


def _payload1(g, ea, amf, ws):
    espec = lambda w: pl.BlockSpec((BE, w), lambda i: (i, 0))
    return pl.pallas_call(
        _payload1_body,
        grid=(E_PAD // BE,),
        in_specs=[espec(DW), espec(9), espec(2), _full_spec((9, DW, 48))],
        out_specs=espec(DW),
        out_shape=jax.ShapeDtypeStruct((E_PAD, DW), jnp.float32),
    )(g, ea, amf, ws)


def _payload2(g, ea, ws):
    espec = lambda w: pl.BlockSpec((BE, w), lambda i: (i, 0))
    return pl.pallas_call(
        _payload2_body,
        grid=(E_PAD // BE,),
        in_specs=[espec(DW), espec(9), _full_spec((9, DW, 48))],
        out_specs=espec(DW),
        out_shape=jax.ShapeDtypeStruct((E_PAD, DW), jnp.float32),
    )(g, ea, ws)


def _update1(part, h, wd, wa):
    nb = N // BN
    return pl.pallas_call(
        _update1_body,
        grid=(nb,),
        in_specs=[
            _node_spec(DW),
            _node_spec(DW, lambda i: (i + nb, 0)),
            _node_spec(DW),
            _full_spec((9, DW, 48)),
            _full_spec((18, 48)),
        ],
        out_specs=[_node_spec(DW), _node_spec(27)],
        out_shape=[jax.ShapeDtypeStruct((N, DW), jnp.float32),
                   jax.ShapeDtypeStruct((N, 27), jnp.float32)],
    )(part, part, h, wd, wa)


def _final(part, h, asv, na, wd, wa, wp1, wp2):
    nb = N // BN
    return pl.pallas_call(
        _final_body,
        grid=(nb,),
        in_specs=[
            _node_spec(DW),
            _node_spec(DW, lambda i: (i + nb, 0)),
            _node_spec(DW),
            _node_spec(27),
            _node_spec(9),
            _full_spec((9, DW, 48)),
            _full_spec((18, 48)),
            _full_spec((9, DW, 48)),
            _full_spec((9, DW, 16)),
        ],
        out_specs=_node_spec(16),
        out_shape=jax.ShapeDtypeStruct((N, 16), jnp.float32),
    )(part, part, h, asv, na, wd, wa, wp1, wp2)


def kernel(x, pos, edge_index, edge_attr, node_attr, batch,
           additional_message_features, W_embed, W_msg0, W_msg1, W_pp1, W_pp2):
    f32 = jnp.float32
    pad_e = E_PAD - E
    srcp = jnp.concatenate([edge_index[0], jnp.zeros((pad_e,), jnp.int32)])
    dstp = jnp.concatenate([edge_index[1], jnp.zeros((pad_e,), jnp.int32)])
    eap = jnp.concatenate([edge_attr, jnp.zeros((pad_e, 9), f32)], axis=0)
    amfp = jnp.concatenate(
        [additional_message_features, jnp.zeros((pad_e, 2), f32)], axis=0)

    def prep_w(w, kpad=None):
        din, _, k = w.shape
        w = jnp.concatenate([w, jnp.zeros((DW - din, 9, k), f32)], axis=0)
        w = jnp.transpose(w, (1, 0, 2))
        if kpad is not None and k < kpad:
            w = jnp.concatenate([w, jnp.zeros((9, DW, kpad - k), f32)], axis=2)
        return w

    we = prep_w(W_embed, kpad=48)
    wd0 = prep_w(W_msg0[:40])
    ws0 = prep_w(W_msg0[40:80])
    wa0 = W_msg0[80:82].reshape(18, 48)
    wd1 = prep_w(W_msg1[:40])
    ws1 = prep_w(W_msg1[40:80])
    wa1 = W_msg1[80:82].reshape(18, 48)
    wp1 = prep_w(W_pp1)
    wp2 = prep_w(W_pp2)

    zrows = jnp.zeros((RCP, DW), f32)

    h0 = _embed(x, node_attr, we)
    g0 = _sc_gather()(h0, srcp)
    pay0 = _payload1(g0, eap, amfp, ws0)
    part0 = _sc_scatter()(pay0, dstp, zrows)
    h1, asv = _update1(part0, h0, wd0, wa0)
    g1 = _sc_gather()(h1, srcp)
    pay1 = _payload2(g1, eap, ws1)
    part1 = _sc_scatter()(pay1, dstp, zrows)
    return _final(part1, h1, asv, node_attr, wd1, wa1, wp1, wp2)
